# baseline (speedup 1.0000x reference)
import functools

import jax
import jax.numpy as jnp
import numpy as np
from jax import lax
from jax.experimental import pallas as pl
from jax.experimental.pallas import tpu as pltpu
from jax.experimental.pallas import tpu_sc as plsc

_B = 128
_IMG = 3 * 224 * 224
_NB_OP = 36
_PAD_OPS = 48
_NW = 32
_IPW = _B // _NW
_CHUNK = 37632
_NCHUNK = _IMG // _CHUNK
_VECS = _CHUNK // 16
_BIG = np.float32(1e30)


def _op_tables():
    c = np.full(_PAD_OPS, -_BIG, np.float32)
    a2 = np.ones(_PAD_OPS, np.float32)
    b2 = np.zeros(_PAD_OPS, np.float32)
    lo = np.full(_PAD_OPS, -_BIG, np.float32)
    hi = np.full(_PAD_OPS, _BIG, np.float32)
    names = ["Identity", "Brightness", "Contrast", "Solarize"]
    for op in range(_NB_OP):
        name = names[op // 9]
        mag = (op % 9 + 1) / 10.0
        if name == "Brightness":
            b2[op] = np.float32(mag)
            lo[op], hi[op] = 0.0, 1.0
        elif name == "Contrast":
            a2[op] = np.float32(1.0 + mag)
            lo[op], hi[op] = 0.0, 1.0
        elif name == "Solarize":
            c[op] = np.float32(mag)
            a2[op] = -1.0
            b2[op] = 1.0
    return np.concatenate([c, a2, b2, lo, hi])


def _select48(tab_ref, base, iota16, op, default):
    acc = jnp.float32(0.0)
    for k in range(3):
        v = tab_ref[pl.ds(base + 16 * k, 16)]
        acc = acc + jnp.sum(jnp.where(iota16 + (16 * k) == op, v, jnp.float32(0.0)))
    return jnp.where(op < _NB_OP, acc, default)


def _sc_body(g_hbm, u_hbm, tab_hbm, x_hbm, out_hbm,
             gbuf, ubuf, tbuf, buf0, buf1, sem0, sem1):
    cid = lax.axis_index("c")
    sid = lax.axis_index("s")
    wid = sid * 2 + cid

    pltpu.sync_copy(g_hbm.at[pl.ds(wid * (_IPW * _PAD_OPS), _IPW * _PAD_OPS)], gbuf)
    pltpu.sync_copy(u_hbm, ubuf)
    pltpu.sync_copy(tab_hbm, tbuf)

    iota16 = lax.iota(jnp.int32, 16)
    bufs = (buf0, buf1)
    sems = (sem0, sem1)

    for j in range(_IPW):
        v0 = gbuf[pl.ds(j * _PAD_OPS, 16)]
        v1 = gbuf[pl.ds(j * _PAD_OPS + 16, 16)]
        v2 = gbuf[pl.ds(j * _PAD_OPS + 32, 16)]
        gmax = jnp.max(jnp.maximum(jnp.maximum(v0, v1), v2))
        big_i = jnp.int32(999)
        i0 = jnp.min(jnp.where(v0 == gmax, iota16, big_i))
        i1 = jnp.min(jnp.where(v1 == gmax, iota16 + 16, big_i))
        i2 = jnp.min(jnp.where(v2 == gmax, iota16 + 32, big_i))
        op = jnp.minimum(jnp.minimum(i0, i1), i2)

        u0 = ubuf[pl.ds(0, 16)]
        u1 = ubuf[pl.ds(16, 16)]
        u2 = ubuf[pl.ds(32, 16)]
        s0 = jnp.min(jnp.where(iota16 == op, u0, jnp.float32(1.0)))
        s1 = jnp.min(jnp.where(iota16 + 16 == op, u1, jnp.float32(1.0)))
        s2 = jnp.min(jnp.where(iota16 + 32 == op, u2, jnp.float32(1.0)))
        coin = jnp.minimum(jnp.minimum(s0, s1), s2) < jnp.float32(0.5)

        c_s = _select48(tbuf, 0, iota16, op, -_BIG)
        a2_s = _select48(tbuf, 48, iota16, op, jnp.float32(1.0))
        b2_s = _select48(tbuf, 96, iota16, op, jnp.float32(0.0))
        lo_s = _select48(tbuf, 144, iota16, op, -_BIG)
        hi_s = _select48(tbuf, 192, iota16, op, _BIG)
        c_p = jnp.where(coin, c_s, -_BIG)
        a2_p = jnp.where(coin, a2_s, jnp.float32(1.0))
        b2_p = jnp.where(coin, b2_s, jnp.float32(0.0))
        lo_p = jnp.where(coin, lo_s, -_BIG)
        hi_p = jnp.where(coin, hi_s, _BIG)

        base = (wid * _IPW + j) * _IMG

        def _in(ch, slot):
            return pltpu.async_copy(
                x_hbm.at[pl.ds(base + ch * _CHUNK, _CHUNK)], bufs[slot], sems[slot])

        cp = _in(0, 0)
        for ch in range(_NCHUNK):
            slot = ch % 2
            cp.wait()
            if ch + 1 < _NCHUNK:
                cp = _in(ch + 1, 1 - slot)
            buf = bufs[slot]

            def body(i, _):
                for k in range(4):
                    v = buf[pl.ds(i * 64 + k * 16, 16)]
                    y = jnp.where(v < c_p, v, a2_p * v + b2_p)
                    y = jnp.minimum(jnp.maximum(y, lo_p), hi_p)
                    buf[pl.ds(i * 64 + k * 16, 16)] = y
                return 0

            lax.fori_loop(0, _VECS // 4, body, 0)
            pltpu.sync_copy(buf, out_hbm.at[pl.ds(base + ch * _CHUNK, _CHUNK)])


@functools.partial(jax.jit, static_argnames=())
def _run(x_flat, g, u, tab):
    mesh = plsc.VectorSubcoreMesh(core_axis_name="c", subcore_axis_name="s")
    f = pl.kernel(
        _sc_body,
        out_type=jax.ShapeDtypeStruct((_B * _IMG,), jnp.float32),
        mesh=mesh,
        compiler_params=pltpu.CompilerParams(needs_layout_passes=False),
        scratch_types=[
            pltpu.VMEM((_IPW * _PAD_OPS,), jnp.float32),
            pltpu.VMEM((_PAD_OPS,), jnp.float32),
            pltpu.VMEM((5 * _PAD_OPS,), jnp.float32),
            pltpu.VMEM((_CHUNK,), jnp.float32),
            pltpu.VMEM((_CHUNK,), jnp.float32),
            pltpu.SemaphoreType.DMA,
            pltpu.SemaphoreType.DMA,
        ],
    )
    return f(g, u, tab, x_flat)


def kernel(x):
    key = jax.random.key(1)
    k = jax.random.fold_in(key, 0)
    g = jax.random.gumbel(jax.random.fold_in(k, 0), (_B, _NB_OP), jnp.float32)
    u = jax.random.uniform(jax.random.fold_in(k, 1), (_NB_OP,), jnp.float32)
    g48 = jnp.concatenate(
        [g, jnp.full((_B, _PAD_OPS - _NB_OP), -_BIG, jnp.float32)], axis=1
    ).reshape(-1)
    u48 = jnp.concatenate(
        [u, jnp.ones((_PAD_OPS - _NB_OP,), jnp.float32)])
    tab = jnp.asarray(_op_tables())
    out = _run(x.reshape(-1), g48, u48, tab)
    return out.reshape(x.shape)


# speedup vs baseline: 1.0359x; 1.0359x over previous
import functools

import jax
import jax.numpy as jnp
import numpy as np
from jax import lax
from jax.experimental import pallas as pl
from jax.experimental.pallas import tpu as pltpu
from jax.experimental.pallas import tpu_sc as plsc

_B = 128
_IMG = 3 * 224 * 224
_NB_OP = 36
_PAD_OPS = 48
_NW = 32
_SL = _IMG // _NW
_SVECS = _SL // 16
_BIG = np.float32(1e30)


def _op_tables():
    c = np.full(_PAD_OPS, -_BIG, np.float32)
    a2 = np.ones(_PAD_OPS, np.float32)
    b2 = np.zeros(_PAD_OPS, np.float32)
    lo = np.full(_PAD_OPS, -_BIG, np.float32)
    hi = np.full(_PAD_OPS, _BIG, np.float32)
    names = ["Identity", "Brightness", "Contrast", "Solarize"]
    for op in range(_NB_OP):
        name = names[op // 9]
        mag = (op % 9 + 1) / 10.0
        if name == "Brightness":
            b2[op] = np.float32(mag)
            lo[op], hi[op] = 0.0, 1.0
        elif name == "Contrast":
            a2[op] = np.float32(1.0 + mag)
            lo[op], hi[op] = 0.0, 1.0
        elif name == "Solarize":
            c[op] = np.float32(mag)
            a2[op] = -1.0
            b2[op] = 1.0
    return np.concatenate([c, a2, b2, lo, hi])


def _sc_body(g_hbm, u_hbm, tab_hbm, x_hbm, out_hbm,
             gbuf, ubuf, tbuf, in0, in1, ou0, ou1,
             si0, si1, so0, so1):
    cid = lax.axis_index("c")
    sid = lax.axis_index("s")
    wid = sid * 2 + cid
    woff = wid * _SL

    pltpu.sync_copy(g_hbm, gbuf)
    pltpu.sync_copy(u_hbm, ubuf)
    pltpu.sync_copy(tab_hbm, tbuf)

    iota16 = lax.iota(jnp.int32, 16)
    ins = (in0, in1)
    ous = (ou0, ou1)
    sis = (si0, si1)
    sos = (so0, so1)

    def issue_in(i, p):
        pltpu.async_copy(x_hbm.at[pl.ds(i * _IMG + woff, _SL)], ins[p], sis[p])

    def issue_out(i, p):
        pltpu.async_copy(ous[p], out_hbm.at[pl.ds(i * _IMG + woff, _SL)], sos[p])

    def wait_in(p):
        pltpu.make_async_copy(x_hbm.at[pl.ds(0, _SL)], ins[p], sis[p]).wait()

    def wait_out(p):
        pltpu.make_async_copy(ous[p], out_hbm.at[pl.ds(0, _SL)], sos[p]).wait()

    def params_for(i):
        v0 = gbuf[pl.ds(i * _PAD_OPS, 16)]
        v1 = gbuf[pl.ds(i * _PAD_OPS + 16, 16)]
        v2 = gbuf[pl.ds(i * _PAD_OPS + 32, 16)]
        gmax = jnp.max(jnp.maximum(jnp.maximum(v0, v1), v2))
        big_i = jnp.int32(999)
        idxv = jnp.minimum(
            jnp.minimum(jnp.where(v0 == gmax, iota16, big_i),
                        jnp.where(v1 == gmax, iota16 + 16, big_i)),
            jnp.where(v2 == gmax, iota16 + 32, big_i))
        op = jnp.min(idxv)

        u0 = ubuf[pl.ds(0, 16)]
        u1 = ubuf[pl.ds(16, 16)]
        u2 = ubuf[pl.ds(32, 16)]
        one = jnp.float32(1.0)
        uv = jnp.minimum(
            jnp.minimum(jnp.where(iota16 == op, u0, one),
                        jnp.where(iota16 + 16 == op, u1, one)),
            jnp.where(iota16 + 32 == op, u2, one))
        coin = jnp.min(uv) < jnp.float32(0.5)

        def sel(base, default):
            z = jnp.float32(0.0)
            acc = (jnp.where(iota16 == op, tbuf[pl.ds(base, 16)], z)
                   + jnp.where(iota16 + 16 == op, tbuf[pl.ds(base + 16, 16)], z)
                   + jnp.where(iota16 + 32 == op, tbuf[pl.ds(base + 32, 16)], z))
            return jnp.where(coin, jnp.sum(acc), default)

        return (sel(0, -_BIG), sel(48, one), sel(96, jnp.float32(0.0)),
                sel(144, -_BIG), sel(192, _BIG))

    issue_in(0, 0)
    issue_in(1, 1)

    def body(t, _):
        for p in (0, 1):
            i = 2 * t + p
            c_p, a2_p, b2_p, lo_p, hi_p = params_for(i)
            wait_in(p)

            @pl.when(i >= 2)
            def _():
                wait_out(p)

            inb, oub = ins[p], ous[p]

            @plsc.parallel_loop(0, _SVECS, unroll=7)
            def _(v):
                xv = inb[pl.ds(v * 16, 16)]
                y = jnp.where(xv < c_p, xv, a2_p * xv + b2_p)
                oub[pl.ds(v * 16, 16)] = jnp.minimum(jnp.maximum(y, lo_p), hi_p)

            @pl.when(i < _B - 2)
            def _():
                issue_in(i + 2, p)

            issue_out(i, p)
        return 0

    lax.fori_loop(0, _B // 2, body, 0)
    wait_out(0)
    wait_out(1)


@jax.jit
def _run(x_flat, g, u, tab):
    mesh = plsc.VectorSubcoreMesh(core_axis_name="c", subcore_axis_name="s")
    f = pl.kernel(
        _sc_body,
        out_type=jax.ShapeDtypeStruct((_B * _IMG,), jnp.float32),
        mesh=mesh,
        compiler_params=pltpu.CompilerParams(needs_layout_passes=False),
        scratch_types=[
            pltpu.VMEM((_B * _PAD_OPS,), jnp.float32),
            pltpu.VMEM((_PAD_OPS,), jnp.float32),
            pltpu.VMEM((5 * _PAD_OPS,), jnp.float32),
            pltpu.VMEM((_SL,), jnp.float32),
            pltpu.VMEM((_SL,), jnp.float32),
            pltpu.VMEM((_SL,), jnp.float32),
            pltpu.VMEM((_SL,), jnp.float32),
            pltpu.SemaphoreType.DMA,
            pltpu.SemaphoreType.DMA,
            pltpu.SemaphoreType.DMA,
            pltpu.SemaphoreType.DMA,
        ],
    )
    return f(g, u, tab, x_flat)


def kernel(x):
    key = jax.random.key(1)
    k = jax.random.fold_in(key, 0)
    g = jax.random.gumbel(jax.random.fold_in(k, 0), (_B, _NB_OP), jnp.float32)
    u = jax.random.uniform(jax.random.fold_in(k, 1), (_NB_OP,), jnp.float32)
    g48 = jnp.concatenate(
        [g, jnp.full((_B, _PAD_OPS - _NB_OP), -_BIG, jnp.float32)], axis=1
    ).reshape(-1)
    u48 = jnp.concatenate(
        [u, jnp.ones((_PAD_OPS - _NB_OP,), jnp.float32)])
    tab = jnp.asarray(_op_tables())
    out = _run(x.reshape(-1), g48, u48, tab)
    return out.reshape(x.shape)


# speedup vs baseline: 1.0373x; 1.0013x over previous
import functools

import jax
import jax.numpy as jnp
import numpy as np
from jax import lax
from jax.experimental import pallas as pl
from jax.experimental.pallas import tpu as pltpu
from jax.experimental.pallas import tpu_sc as plsc

_B = 128
_IMG = 3 * 224 * 224
_NB_OP = 36
_PAD_OPS = 48
_NW = 32
_SL = _IMG // _NW
_SVECS = _SL // 16
_BIG = np.float32(1e30)


def _op_tables():
    c = np.full(_PAD_OPS, -_BIG, np.float32)
    a2 = np.ones(_PAD_OPS, np.float32)
    b2 = np.zeros(_PAD_OPS, np.float32)
    lo = np.full(_PAD_OPS, -_BIG, np.float32)
    hi = np.full(_PAD_OPS, _BIG, np.float32)
    names = ["Identity", "Brightness", "Contrast", "Solarize"]
    for op in range(_NB_OP):
        name = names[op // 9]
        mag = (op % 9 + 1) / 10.0
        if name == "Brightness":
            b2[op] = np.float32(mag)
            lo[op], hi[op] = 0.0, 1.0
        elif name == "Contrast":
            a2[op] = np.float32(1.0 + mag)
            lo[op], hi[op] = 0.0, 1.0
        elif name == "Solarize":
            c[op] = np.float32(mag)
            a2[op] = -1.0
            b2[op] = 1.0
    return np.concatenate([c, a2, b2, lo, hi])


def _sc_body(g_hbm, u_hbm, tab_hbm, x_hbm, out_hbm,
             gbuf, ubuf, tbuf, in0, in1, ou0, ou1,
             si0, si1, so0, so1):
    cid = lax.axis_index("c")
    sid = lax.axis_index("s")
    wid = sid * 2 + cid
    woff = wid * _SL

    pltpu.sync_copy(g_hbm, gbuf)
    pltpu.sync_copy(u_hbm, ubuf)
    pltpu.sync_copy(tab_hbm, tbuf)

    iota16 = lax.iota(jnp.int32, 16)
    ins = (in0, in1)
    ous = (ou0, ou1)
    sis = (si0, si1)
    sos = (so0, so1)

    def issue_in(i, p):
        pltpu.async_copy(x_hbm.at[pl.ds(i * _IMG + woff, _SL)], ins[p], sis[p])

    def issue_out(i, p):
        pltpu.async_copy(ous[p], out_hbm.at[pl.ds(i * _IMG + woff, _SL)], sos[p])

    def wait_in(p):
        pltpu.make_async_copy(x_hbm.at[pl.ds(0, _SL)], ins[p], sis[p]).wait()

    def wait_out(p):
        pltpu.make_async_copy(ous[p], out_hbm.at[pl.ds(0, _SL)], sos[p]).wait()

    def params_for(i):
        v0 = gbuf[pl.ds(i * _PAD_OPS, 16)]
        v1 = gbuf[pl.ds(i * _PAD_OPS + 16, 16)]
        v2 = gbuf[pl.ds(i * _PAD_OPS + 32, 16)]
        gmax = jnp.max(jnp.maximum(jnp.maximum(v0, v1), v2))
        big_i = jnp.int32(999)
        idxv = jnp.minimum(
            jnp.minimum(jnp.where(v0 == gmax, iota16, big_i),
                        jnp.where(v1 == gmax, iota16 + 16, big_i)),
            jnp.where(v2 == gmax, iota16 + 32, big_i))
        op = jnp.min(idxv)

        u0 = ubuf[pl.ds(0, 16)]
        u1 = ubuf[pl.ds(16, 16)]
        u2 = ubuf[pl.ds(32, 16)]
        one = jnp.float32(1.0)
        uv = jnp.minimum(
            jnp.minimum(jnp.where(iota16 == op, u0, one),
                        jnp.where(iota16 + 16 == op, u1, one)),
            jnp.where(iota16 + 32 == op, u2, one))
        coin = jnp.min(uv) < jnp.float32(0.5)

        def sel(base, default):
            z = jnp.float32(0.0)
            acc = (jnp.where(iota16 == op, tbuf[pl.ds(base, 16)], z)
                   + jnp.where(iota16 + 16 == op, tbuf[pl.ds(base + 16, 16)], z)
                   + jnp.where(iota16 + 32 == op, tbuf[pl.ds(base + 32, 16)], z))
            return jnp.where(coin, jnp.sum(acc), default)

        return (sel(0, -_BIG), sel(48, one), sel(96, jnp.float32(0.0)),
                sel(144, -_BIG), sel(192, _BIG))

    issue_in(0, 0)
    issue_in(1, 1)

    def body(t, _):
        for p in (0, 1):
            i = 2 * t + p
            c_p, a2_p, b2_p, lo_p, hi_p = params_for(i)
            wait_in(p)

            @pl.when(i >= 2)
            def _():
                wait_out(p)

            inb, oub = ins[p], ous[p]

            @plsc.parallel_loop(0, _SVECS, unroll=7)
            def _(v):
                xv = inb[pl.ds(v * 16, 16)]
                y = jnp.where(xv < c_p, xv, a2_p * xv + b2_p)
                oub[pl.ds(v * 16, 16)] = jnp.minimum(jnp.maximum(y, lo_p), hi_p)

            @pl.when(i < _B - 2)
            def _():
                issue_in(i + 2, p)

            issue_out(i, p)
        return 0

    lax.fori_loop(0, _B // 2, body, 0)
    wait_out(0)
    wait_out(1)


@jax.jit
def _run(x_flat, g, u, tab):
    mesh = plsc.VectorSubcoreMesh(core_axis_name="c", subcore_axis_name="s")
    f = pl.kernel(
        _sc_body,
        out_type=jax.ShapeDtypeStruct((_B * _IMG,), jnp.float32),
        mesh=mesh,
        compiler_params=pltpu.CompilerParams(
            needs_layout_passes=False, skip_device_barrier=True),
        scratch_types=[
            pltpu.VMEM((_B * _PAD_OPS,), jnp.float32),
            pltpu.VMEM((_PAD_OPS,), jnp.float32),
            pltpu.VMEM((5 * _PAD_OPS,), jnp.float32),
            pltpu.VMEM((_SL,), jnp.float32),
            pltpu.VMEM((_SL,), jnp.float32),
            pltpu.VMEM((_SL,), jnp.float32),
            pltpu.VMEM((_SL,), jnp.float32),
            pltpu.SemaphoreType.DMA,
            pltpu.SemaphoreType.DMA,
            pltpu.SemaphoreType.DMA,
            pltpu.SemaphoreType.DMA,
        ],
    )
    return f(g, u, tab, x_flat)


def kernel(x):
    key = jax.random.key(1)
    k = jax.random.fold_in(key, 0)
    g = jax.random.gumbel(jax.random.fold_in(k, 0), (_B, _NB_OP), jnp.float32)
    u = jax.random.uniform(jax.random.fold_in(k, 1), (_NB_OP,), jnp.float32)
    g48 = jnp.concatenate(
        [g, jnp.full((_B, _PAD_OPS - _NB_OP), -_BIG, jnp.float32)], axis=1
    ).reshape(-1)
    u48 = jnp.concatenate(
        [u, jnp.ones((_PAD_OPS - _NB_OP,), jnp.float32)])
    tab = jnp.asarray(_op_tables())
    out = _run(x.reshape(-1), g48, u48, tab)
    return out.reshape(x.shape)


# speedup vs baseline: 1.9138x; 1.8450x over previous
import functools

import jax
import jax.numpy as jnp
import numpy as np
from jax import lax
from jax.experimental import pallas as pl
from jax.experimental.pallas import tpu as pltpu
from jax.experimental.pallas import tpu_sc as plsc

_B = 128
_IMG = 3 * 224 * 224
_ROWS = 672
_COLS = 224
_NB_OP = 36
_PAD_OPS = 48
_NW = 32
_CROWS = 96
_NCH = _ROWS // _CROWS
_UPW = _B * _NCH // _NW
_CVECS = _COLS // 16
_BIG = np.float32(1e30)


def _op_tables():
    c = np.full(_PAD_OPS, -_BIG, np.float32)
    a2 = np.ones(_PAD_OPS, np.float32)
    b2 = np.zeros(_PAD_OPS, np.float32)
    lo = np.full(_PAD_OPS, -_BIG, np.float32)
    hi = np.full(_PAD_OPS, _BIG, np.float32)
    names = ["Identity", "Brightness", "Contrast", "Solarize"]
    for op in range(_NB_OP):
        name = names[op // 9]
        mag = (op % 9 + 1) / 10.0
        if name == "Brightness":
            b2[op] = np.float32(mag)
            lo[op], hi[op] = 0.0, 1.0
        elif name == "Contrast":
            a2[op] = np.float32(1.0 + mag)
            lo[op], hi[op] = 0.0, 1.0
        elif name == "Solarize":
            c[op] = np.float32(mag)
            a2[op] = -1.0
            b2[op] = 1.0
    return np.concatenate([c, a2, b2, lo, hi])


def _sc_body(g_hbm, u_hbm, tab_hbm, x_hbm, out_hbm,
             gbuf, ubuf, tbuf, in0, in1, ou0, ou1,
             si0, si1, so0, so1):
    cid = lax.axis_index("c")
    sid = lax.axis_index("s")
    wid = sid * 2 + cid
    ubase = wid * _UPW

    pltpu.sync_copy(g_hbm, gbuf)
    pltpu.sync_copy(u_hbm, ubuf)
    pltpu.sync_copy(tab_hbm, tbuf)

    iota16 = lax.iota(jnp.int32, 16)
    ins = (in0, in1)
    ous = (ou0, ou1)
    sis = (si0, si1)
    sos = (so0, so1)

    def unit(t):
        u = ubase + t
        return u // _NCH, (u % _NCH) * _CROWS

    def issue_in(t, p):
        i, r = unit(t)
        pltpu.async_copy(x_hbm.at[i, pl.ds(r, _CROWS), :], ins[p], sis[p])

    def issue_out(t, p):
        i, r = unit(t)
        pltpu.async_copy(ous[p], out_hbm.at[i, pl.ds(r, _CROWS), :], sos[p])

    def wait_in(p):
        pltpu.make_async_copy(x_hbm.at[0, pl.ds(0, _CROWS), :], ins[p], sis[p]).wait()

    def wait_out(p):
        pltpu.make_async_copy(ous[p], out_hbm.at[0, pl.ds(0, _CROWS), :], sos[p]).wait()

    def params_for(i):
        v0 = gbuf[pl.ds(i * _PAD_OPS, 16)]
        v1 = gbuf[pl.ds(i * _PAD_OPS + 16, 16)]
        v2 = gbuf[pl.ds(i * _PAD_OPS + 32, 16)]
        gmax = jnp.max(jnp.maximum(jnp.maximum(v0, v1), v2))
        big_i = jnp.int32(999)
        idxv = jnp.minimum(
            jnp.minimum(jnp.where(v0 == gmax, iota16, big_i),
                        jnp.where(v1 == gmax, iota16 + 16, big_i)),
            jnp.where(v2 == gmax, iota16 + 32, big_i))
        op = jnp.min(idxv)

        u0 = ubuf[pl.ds(0, 16)]
        u1 = ubuf[pl.ds(16, 16)]
        u2 = ubuf[pl.ds(32, 16)]
        one = jnp.float32(1.0)
        uv = jnp.minimum(
            jnp.minimum(jnp.where(iota16 == op, u0, one),
                        jnp.where(iota16 + 16 == op, u1, one)),
            jnp.where(iota16 + 32 == op, u2, one))
        coin = jnp.min(uv) < jnp.float32(0.5)

        def sel(base, default):
            z = jnp.float32(0.0)
            acc = (jnp.where(iota16 == op, tbuf[pl.ds(base, 16)], z)
                   + jnp.where(iota16 + 16 == op, tbuf[pl.ds(base + 16, 16)], z)
                   + jnp.where(iota16 + 32 == op, tbuf[pl.ds(base + 32, 16)], z))
            return jnp.where(coin, jnp.sum(acc), default)

        return (sel(0, -_BIG), sel(48, one), sel(96, jnp.float32(0.0)),
                sel(144, -_BIG), sel(192, _BIG))

    issue_in(0, 0)
    issue_in(1, 1)

    def body(h, _):
        for p in (0, 1):
            t = 2 * h + p
            img = (ubase + t) // _NCH
            c_p, a2_p, b2_p, lo_p, hi_p = params_for(img)
            wait_in(p)

            @pl.when(t >= 2)
            def _():
                wait_out(p)

            inb, oub = ins[p], ous[p]

            @plsc.parallel_loop(0, _CROWS, unroll=2)
            def _(r):
                for k in range(_CVECS):
                    xv = inb[r, pl.ds(k * 16, 16)]
                    y = jnp.where(xv < c_p, xv, a2_p * xv + b2_p)
                    oub[r, pl.ds(k * 16, 16)] = jnp.minimum(
                        jnp.maximum(y, lo_p), hi_p)

            @pl.when(t < _UPW - 2)
            def _():
                issue_in(t + 2, p)

            issue_out(t, p)
        return 0

    lax.fori_loop(0, _UPW // 2, body, 0)
    wait_out(0)
    wait_out(1)


@jax.jit
def _run(x_flat, g, u, tab):
    mesh = plsc.VectorSubcoreMesh(core_axis_name="c", subcore_axis_name="s")
    f = pl.kernel(
        _sc_body,
        out_type=jax.ShapeDtypeStruct((_B, _ROWS, _COLS), jnp.float32),
        mesh=mesh,
        compiler_params=pltpu.CompilerParams(
            needs_layout_passes=False, skip_device_barrier=True),
        scratch_types=[
            pltpu.VMEM((_B * _PAD_OPS,), jnp.float32),
            pltpu.VMEM((_PAD_OPS,), jnp.float32),
            pltpu.VMEM((5 * _PAD_OPS,), jnp.float32),
            pltpu.VMEM((_CROWS, _COLS), jnp.float32),
            pltpu.VMEM((_CROWS, _COLS), jnp.float32),
            pltpu.VMEM((_CROWS, _COLS), jnp.float32),
            pltpu.VMEM((_CROWS, _COLS), jnp.float32),
            pltpu.SemaphoreType.DMA,
            pltpu.SemaphoreType.DMA,
            pltpu.SemaphoreType.DMA,
            pltpu.SemaphoreType.DMA,
        ],
    )
    return f(g, u, tab, x_flat)


def kernel(x):
    key = jax.random.key(1)
    k = jax.random.fold_in(key, 0)
    g = jax.random.gumbel(jax.random.fold_in(k, 0), (_B, _NB_OP), jnp.float32)
    u = jax.random.uniform(jax.random.fold_in(k, 1), (_NB_OP,), jnp.float32)
    g48 = jnp.concatenate(
        [g, jnp.full((_B, _PAD_OPS - _NB_OP), -_BIG, jnp.float32)], axis=1
    ).reshape(-1)
    u48 = jnp.concatenate(
        [u, jnp.ones((_PAD_OPS - _NB_OP,), jnp.float32)])
    tab = jnp.asarray(_op_tables())
    out = _run(x.reshape(_B, _ROWS, _COLS), g48, u48, tab)
    return out.reshape(x.shape)


# speedup vs baseline: 1.9207x; 1.0036x over previous
import functools

import jax
import jax.numpy as jnp
import numpy as np
from jax import lax
from jax.experimental import pallas as pl
from jax.experimental.pallas import tpu as pltpu
from jax.experimental.pallas import tpu_sc as plsc

_B = 128
_IMG = 3 * 224 * 224
_ROWS = 672
_COLS = 224
_NB_OP = 36
_PAD_OPS = 48
_NW = 32
_CROWS = 96
_NCH = _ROWS // _CROWS
_UPW = _B * _NCH // _NW
_CVECS = _COLS // 16
_BIG = np.float32(1e30)


def _op_tables():
    c = np.full(_PAD_OPS, -_BIG, np.float32)
    a2 = np.ones(_PAD_OPS, np.float32)
    b2 = np.zeros(_PAD_OPS, np.float32)
    lo = np.full(_PAD_OPS, -_BIG, np.float32)
    hi = np.full(_PAD_OPS, _BIG, np.float32)
    names = ["Identity", "Brightness", "Contrast", "Solarize"]
    for op in range(_NB_OP):
        name = names[op // 9]
        mag = (op % 9 + 1) / 10.0
        if name == "Brightness":
            b2[op] = np.float32(mag)
            lo[op], hi[op] = 0.0, 1.0
        elif name == "Contrast":
            a2[op] = np.float32(1.0 + mag)
            lo[op], hi[op] = 0.0, 1.0
        elif name == "Solarize":
            c[op] = np.float32(mag)
            a2[op] = -1.0
            b2[op] = 1.0
    return np.concatenate([c, a2, b2, lo, hi])


def _sc_body(g_hbm, u_hbm, tab_hbm, x_hbm, out_hbm,
             gbuf, ubuf, tbuf, in0, in1, ou0, ou1,
             si0, si1, so0, so1):
    cid = lax.axis_index("c")
    sid = lax.axis_index("s")
    wid = sid * 2 + cid
    ubase = wid * _UPW

    pltpu.sync_copy(g_hbm, gbuf)
    pltpu.sync_copy(u_hbm, ubuf)
    pltpu.sync_copy(tab_hbm, tbuf)

    iota16 = lax.iota(jnp.int32, 16)
    ins = (in0, in1)
    ous = (ou0, ou1)
    sis = (si0, si1)
    sos = (so0, so1)

    def unit(t):
        u = ubase + t
        return u // _NCH, (u % _NCH) * _CROWS

    def issue_in(t, p):
        i, r = unit(t)
        pltpu.async_copy(x_hbm.at[i, pl.ds(r, _CROWS), :], ins[p], sis[p])

    def issue_out(t, p):
        i, r = unit(t)
        pltpu.async_copy(ous[p], out_hbm.at[i, pl.ds(r, _CROWS), :], sos[p])

    def wait_in(p):
        pltpu.make_async_copy(x_hbm.at[0, pl.ds(0, _CROWS), :], ins[p], sis[p]).wait()

    def wait_out(p):
        pltpu.make_async_copy(ous[p], out_hbm.at[0, pl.ds(0, _CROWS), :], sos[p]).wait()

    def params_for(i):
        v0 = gbuf[pl.ds(i * _PAD_OPS, 16)]
        v1 = gbuf[pl.ds(i * _PAD_OPS + 16, 16)]
        v2 = gbuf[pl.ds(i * _PAD_OPS + 32, 16)]
        gmax = jnp.max(jnp.maximum(jnp.maximum(v0, v1), v2))
        big_i = jnp.int32(999)
        idxv = jnp.minimum(
            jnp.minimum(jnp.where(v0 == gmax, iota16, big_i),
                        jnp.where(v1 == gmax, iota16 + 16, big_i)),
            jnp.where(v2 == gmax, iota16 + 32, big_i))
        op = jnp.min(idxv)

        u0 = ubuf[pl.ds(0, 16)]
        u1 = ubuf[pl.ds(16, 16)]
        u2 = ubuf[pl.ds(32, 16)]
        one = jnp.float32(1.0)
        uv = jnp.minimum(
            jnp.minimum(jnp.where(iota16 == op, u0, one),
                        jnp.where(iota16 + 16 == op, u1, one)),
            jnp.where(iota16 + 32 == op, u2, one))
        coin = jnp.min(uv) < jnp.float32(0.5)

        def sel(base, default):
            z = jnp.float32(0.0)
            acc = (jnp.where(iota16 == op, tbuf[pl.ds(base, 16)], z)
                   + jnp.where(iota16 + 16 == op, tbuf[pl.ds(base + 16, 16)], z)
                   + jnp.where(iota16 + 32 == op, tbuf[pl.ds(base + 32, 16)], z))
            return jnp.where(coin, jnp.sum(acc), default)

        return (sel(0, -_BIG), sel(48, one), sel(96, jnp.float32(0.0)),
                sel(144, -_BIG), sel(192, _BIG))

    issue_in(0, 0)
    issue_in(1, 1)

    def body(h, _):
        for p in (0, 1):
            t = 2 * h + p
            img = (ubase + t) // _NCH
            c_p, a2_p, b2_p, lo_p, hi_p = params_for(img)
            wait_in(p)

            @pl.when(t >= 2)
            def _():
                wait_out(p)

            inb, oub = ins[p], ous[p]

            @plsc.parallel_loop(0, _CROWS, unroll=2)
            def _(r):
                for k in range(_CVECS):
                    xv = inb[r, pl.ds(k * 16, 16)]
                    y = jnp.where(xv < c_p, xv, a2_p * xv + b2_p)
                    oub[r, pl.ds(k * 16, 16)] = jnp.minimum(
                        jnp.maximum(y, lo_p), hi_p)

            @pl.when(t < _UPW - 2)
            def _():
                issue_in(t + 2, p)

            issue_out(t, p)
        return 0

    lax.fori_loop(0, _UPW // 2, body, 0)
    wait_out(0)
    wait_out(1)


@jax.jit
def _run(x_flat, g, u, tab):
    mesh = plsc.VectorSubcoreMesh(core_axis_name="c", subcore_axis_name="s")
    f = pl.kernel(
        _sc_body,
        out_type=jax.ShapeDtypeStruct((_B, _ROWS, _COLS), jnp.float32),
        mesh=mesh,
        compiler_params=pltpu.CompilerParams(
            needs_layout_passes=False, skip_device_barrier=True,
            use_tc_tiling_on_sc=True),
        scratch_types=[
            pltpu.VMEM((_B * _PAD_OPS,), jnp.float32),
            pltpu.VMEM((_PAD_OPS,), jnp.float32),
            pltpu.VMEM((5 * _PAD_OPS,), jnp.float32),
            pltpu.VMEM((_CROWS, _COLS), jnp.float32),
            pltpu.VMEM((_CROWS, _COLS), jnp.float32),
            pltpu.VMEM((_CROWS, _COLS), jnp.float32),
            pltpu.VMEM((_CROWS, _COLS), jnp.float32),
            pltpu.SemaphoreType.DMA,
            pltpu.SemaphoreType.DMA,
            pltpu.SemaphoreType.DMA,
            pltpu.SemaphoreType.DMA,
        ],
    )
    return f(g, u, tab, x_flat)


def kernel(x):
    key = jax.random.key(1)
    k = jax.random.fold_in(key, 0)
    g = jax.random.gumbel(jax.random.fold_in(k, 0), (_B, _NB_OP), jnp.float32)
    u = jax.random.uniform(jax.random.fold_in(k, 1), (_NB_OP,), jnp.float32)
    g48 = jnp.concatenate(
        [g, jnp.full((_B, _PAD_OPS - _NB_OP), -_BIG, jnp.float32)], axis=1
    ).reshape(-1)
    u48 = jnp.concatenate(
        [u, jnp.ones((_PAD_OPS - _NB_OP,), jnp.float32)])
    tab = jnp.asarray(_op_tables())
    out = _run(x.reshape(_B, _ROWS, _COLS), g48, u48, tab)
    return out.reshape(x.shape)
